# baseline (speedup 1.0000x reference)
import functools

import jax
import jax.numpy as jnp
from jax import lax
from jax.experimental import pallas as pl
from jax.experimental.pallas import tpu as pltpu
from jax.experimental.pallas import tpu_sc as plsc

MARGIN = 0.5
WEIGHT = 1.0

_B = 4096
_D = 128
_NW = 32
_RPW = _B // _NW
_NCH = _RPW // 16
_BE = _B + 16
_TBL = _B + 16
_SCMESH = plsc.VectorSubcoreMesh(core_axis_name="c", subcore_axis_name="s")


def _prep_body(semt_ref, feat_ref, fn_ref, krow_ref):
    f = feat_ref[...]
    ss = jnp.sum(f * f, axis=1, keepdims=True)
    norm = jnp.maximum(jnp.sqrt(ss), 1e-12)
    fn_ref[0:_B, :] = f / norm
    fn_ref[_B:_BE, :] = jnp.zeros((_BE - _B, _D), jnp.float32)
    st = semt_ref[...]
    krow_ref[...] = (
        ((st[0:1, :] * 8 + st[1:2, :]) * 8 + st[2:3, :]) * 8 + st[3:4, :]
    )


def _tc_prep(sem_t, feat):
    return pl.pallas_call(
        _prep_body,
        in_specs=[
            pl.BlockSpec((4, _B), lambda: (0, 0)),
            pl.BlockSpec((_B, _D), lambda: (0, 0)),
        ],
        out_specs=[
            pl.BlockSpec((_BE, _D), lambda: (0, 0)),
            pl.BlockSpec((1, _B), lambda: (0, 0)),
        ],
        out_shape=[
            jax.ShapeDtypeStruct((_BE, _D), jnp.float32),
            jax.ShapeDtypeStruct((1, _B), jnp.int32),
        ],
    )(sem_t, feat)


@functools.partial(
    pl.kernel,
    mesh=_SCMESH,
    out_type=jax.ShapeDtypeStruct((_TBL,), jnp.int32),
    scratch_types=[
        pltpu.VMEM((_RPW,), jnp.int32),
        pltpu.VMEM((_RPW,), jnp.int32),
        pltpu.SemaphoreType.DMA,
    ],
)
def _sc_init(keys_hbm, rid_hbm, tbl_out, kown, rown, sem):
    core = lax.axis_index("c")
    sub = lax.axis_index("s")
    base = (sub * 2 + core) * _RPW
    pltpu.sync_copy(keys_hbm.at[pl.ds(base, _RPW)], kown)
    pltpu.sync_copy(rid_hbm.at[pl.ds(base, _RPW)], rown)
    pltpu.async_copy(rown, tbl_out.at[kown], sem).wait()


@functools.partial(
    pl.kernel,
    mesh=_SCMESH,
    out_type=[
        jax.ShapeDtypeStruct((_B,), jnp.int32),
        jax.ShapeDtypeStruct((_NW * 16,), jnp.int32),
        jax.ShapeDtypeStruct((_B, _D), jnp.float32),
        jax.ShapeDtypeStruct((_TBL,), jnp.int32),
    ],
    scratch_types=[
        pltpu.VMEM((_RPW,), jnp.int32),
        pltpu.VMEM((_RPW,), jnp.int32),
        pltpu.VMEM((_RPW,), jnp.int32),
        pltpu.VMEM((_RPW,), jnp.int32),
        pltpu.VMEM((_RPW,), jnp.int32),
        pltpu.VMEM((_RPW,), jnp.int32),
        pltpu.VMEM((_RPW,), jnp.int32),
        pltpu.VMEM((16,), jnp.int32),
        pltpu.VMEM((_RPW, _D), jnp.float32),
        pltpu.SemaphoreType.DMA,
    ],
)
def _sc_round(fn_hbm, keys_hbm, rid_hbm, act_hbm, tbl_in,
              actout, counts_out, partout, tbl_out,
              kown, aown, rown, win, nact, cidx, pidx, cnt_v, rows_v, sem):
    core = lax.axis_index("c")
    sub = lax.axis_index("s")
    wid = sub * 2 + core
    base = wid * _RPW

    pltpu.sync_copy(keys_hbm.at[pl.ds(base, _RPW)], kown)
    pltpu.sync_copy(act_hbm.at[pl.ds(base, _RPW)], aown)
    pltpu.sync_copy(rid_hbm.at[pl.ds(base, _RPW)], rown)

    pltpu.async_copy(tbl_in.at[kown], win, sem).wait()

    cnt = jnp.zeros((16,), jnp.int32)
    one16 = jnp.full((16,), 1, jnp.int32)
    zero16 = jnp.zeros((16,), jnp.int32)
    sent16 = jnp.full((16,), _B, jnp.int32)
    for c in range(_NCH):
        off = c * 16
        k16 = kown[pl.ds(off, 16)]
        a16 = aown[pl.ds(off, 16)]
        r16 = rown[pl.ds(off, 16)]
        w16 = win[pl.ds(off, 16)]
        pair = (a16 != 0) & (w16 != r16)
        pairi = jnp.where(pair, one16, zero16)
        nact[pl.ds(off, 16)] = pairi
        pidx[pl.ds(off, 16)] = jnp.where(pair, w16, sent16)
        cidx[pl.ds(off, 16)] = jnp.where(pair, k16, sent16)
        cnt = cnt + pairi
    cnt_v[...] = cnt

    pltpu.sync_copy(nact, actout.at[pl.ds(base, _RPW)])
    pltpu.sync_copy(cnt_v, counts_out.at[pl.ds(wid * 16, 16)])

    pltpu.async_copy(rown, tbl_out.at[cidx], sem).wait()

    pltpu.async_copy(fn_hbm.at[pidx], rows_v, sem).wait()
    pltpu.sync_copy(rows_v, partout.at[pl.ds(base, _RPW)])


def _dot_body(fn_ref, part_ref, cnt_ref, sum_ref, cnt_out_ref):
    x = fn_ref[0:_B, :]
    p = part_ref[...]
    dots = jnp.sum(x * p, axis=1)
    hinge = jnp.maximum(dots - MARGIN, 0.0)
    sum_ref[...] = jnp.broadcast_to(jnp.sum(hinge), (1, 128))
    cnt_out_ref[...] = jnp.broadcast_to(jnp.sum(cnt_ref[...]), (1, 128))


def _tc_dots(fn_ext, partfeat, counts):
    return pl.pallas_call(
        _dot_body,
        in_specs=[
            pl.BlockSpec((_BE, _D), lambda: (0, 0)),
            pl.BlockSpec((_B, _D), lambda: (0, 0)),
            pl.BlockSpec((4, 128), lambda: (0, 0)),
        ],
        out_specs=[
            pl.BlockSpec((1, 128), lambda: (0, 0)),
            pl.BlockSpec((1, 128), lambda: (0, 0)),
        ],
        out_shape=[
            jax.ShapeDtypeStruct((1, 128), jnp.float32),
            jax.ShapeDtypeStruct((1, 128), jnp.int32),
        ],
    )(fn_ext, partfeat, counts)


@jax.jit
def kernel(sem_ids, encoded_features):
    fn_ext, krow = _tc_prep(sem_ids.T, encoded_features)
    keys = krow.reshape(_B)
    rowids = jnp.arange(_B, dtype=jnp.int32)
    tbl0 = _sc_init(keys, rowids)

    def _cond(state):
        return state[4]

    def _body(state):
        act, tbl, total, count, _ = state
        nact, counts, partfeat, ntbl = _sc_round(
            fn_ext, keys, rowids, act, tbl)
        rsum, rcnt = _tc_dots(fn_ext, partfeat, counts.reshape(4, 128))
        pairs = rcnt[0, 0]
        return (nact, ntbl, total + rsum[0, 0], count + pairs, pairs > 0)

    act0 = jnp.ones((_B,), jnp.int32)
    init = (act0, tbl0, jnp.float32(0.0), jnp.int32(0), jnp.bool_(True))
    _, _, total, count, _ = lax.while_loop(_cond, _body, init)
    cf = count.astype(jnp.float32)
    mean = WEIGHT * total / jnp.maximum(cf, 1.0)
    return jnp.where(count > 0, mean, 0.0)


# speedup vs baseline: 1.0004x; 1.0004x over previous
import functools

import jax
import jax.numpy as jnp
from jax import lax
from jax.experimental import pallas as pl
from jax.experimental.pallas import tpu as pltpu
from jax.experimental.pallas import tpu_sc as plsc

MARGIN = 0.5
WEIGHT = 1.0

_B = 4096
_D = 128
_NW = 32
_RPW = _B // _NW
_NCH = _RPW // 16
_BE = _B + 16
_TBL = _B + 16
_SCMESH = plsc.VectorSubcoreMesh(core_axis_name="c", subcore_axis_name="s")


def _prep_body(semt_ref, feat_ref, fn_ref, krow_ref):
    f = feat_ref[...]
    ss = jnp.sum(f * f, axis=1, keepdims=True)
    norm = jnp.maximum(jnp.sqrt(ss), 1e-12)
    fn_ref[0:_B, :] = f / norm
    fn_ref[_B:_BE, :] = jnp.zeros((_BE - _B, _D), jnp.float32)
    st = semt_ref[...]
    krow_ref[...] = (
        ((st[0:1, :] * 8 + st[1:2, :]) * 8 + st[2:3, :]) * 8 + st[3:4, :]
    )


def _tc_prep(sem_t, feat):
    return pl.pallas_call(
        _prep_body,
        in_specs=[
            pl.BlockSpec((4, _B), lambda: (0, 0)),
            pl.BlockSpec((_B, _D), lambda: (0, 0)),
        ],
        out_specs=[
            pl.BlockSpec((_BE, _D), lambda: (0, 0)),
            pl.BlockSpec((1, _B), lambda: (0, 0)),
        ],
        out_shape=[
            jax.ShapeDtypeStruct((_BE, _D), jnp.float32),
            jax.ShapeDtypeStruct((1, _B), jnp.int32),
        ],
    )(sem_t, feat)


@functools.partial(
    pl.kernel,
    mesh=_SCMESH,
    out_type=jax.ShapeDtypeStruct((_TBL,), jnp.int32),
    scratch_types=[
        pltpu.VMEM((_RPW,), jnp.int32),
        pltpu.VMEM((_RPW,), jnp.int32),
        pltpu.SemaphoreType.DMA,
    ],
)
def _sc_init(keys_hbm, rid_hbm, tbl_out, kown, rown, sem):
    core = lax.axis_index("c")
    sub = lax.axis_index("s")
    base = (sub * 2 + core) * _RPW
    pltpu.sync_copy(keys_hbm.at[pl.ds(base, _RPW)], kown)
    pltpu.sync_copy(rid_hbm.at[pl.ds(base, _RPW)], rown)
    pltpu.async_copy(rown, tbl_out.at[kown], sem).wait()


@functools.partial(
    pl.kernel,
    mesh=_SCMESH,
    out_type=[
        jax.ShapeDtypeStruct((_B,), jnp.int32),
        jax.ShapeDtypeStruct((_NW * 16,), jnp.int32),
        jax.ShapeDtypeStruct((_B, _D), jnp.float32),
        jax.ShapeDtypeStruct((_TBL,), jnp.int32),
    ],
    scratch_types=[
        pltpu.VMEM((_RPW,), jnp.int32),
        pltpu.VMEM((_RPW,), jnp.int32),
        pltpu.VMEM((_RPW,), jnp.int32),
        pltpu.VMEM((_RPW,), jnp.int32),
        pltpu.VMEM((_RPW,), jnp.int32),
        pltpu.VMEM((_RPW,), jnp.int32),
        pltpu.VMEM((_RPW,), jnp.int32),
        pltpu.VMEM((16,), jnp.int32),
        pltpu.VMEM((_RPW, _D), jnp.float32),
        pltpu.SemaphoreType.DMA,
    ],
)
def _sc_round(fn_hbm, keys_hbm, rid_hbm, act_hbm, tbl_in,
              actout, counts_out, partout, tbl_out,
              kown, aown, rown, win, nact, cidx, pidx, cnt_v, rows_v, sem):
    core = lax.axis_index("c")
    sub = lax.axis_index("s")
    wid = sub * 2 + core
    base = wid * _RPW

    pltpu.sync_copy(keys_hbm.at[pl.ds(base, _RPW)], kown)
    pltpu.sync_copy(act_hbm.at[pl.ds(base, _RPW)], aown)
    pltpu.sync_copy(rid_hbm.at[pl.ds(base, _RPW)], rown)

    wh = [
        pltpu.async_copy(
            tbl_in.at[kown.at[pl.ds(c * 16, 16)]],
            win.at[pl.ds(c * 16, 16)], sem)
        for c in range(_NCH)
    ]
    for h in wh:
        h.wait()

    cnt = jnp.zeros((16,), jnp.int32)
    one16 = jnp.full((16,), 1, jnp.int32)
    zero16 = jnp.zeros((16,), jnp.int32)
    sent16 = jnp.full((16,), _B, jnp.int32)
    for c in range(_NCH):
        off = c * 16
        k16 = kown[pl.ds(off, 16)]
        a16 = aown[pl.ds(off, 16)]
        r16 = rown[pl.ds(off, 16)]
        w16 = win[pl.ds(off, 16)]
        pair = (a16 != 0) & (w16 != r16)
        pairi = jnp.where(pair, one16, zero16)
        nact[pl.ds(off, 16)] = pairi
        pidx[pl.ds(off, 16)] = jnp.where(pair, w16, sent16)
        cidx[pl.ds(off, 16)] = jnp.where(pair, k16, sent16)
        cnt = cnt + pairi
    cnt_v[...] = cnt

    sh = pltpu.async_copy(rown, tbl_out.at[cidx], sem)
    gh = [
        pltpu.async_copy(
            fn_hbm.at[pidx.at[pl.ds(c * 16, 16)]],
            rows_v.at[pl.ds(c * 16, 16)], sem)
        for c in range(_NCH)
    ]
    pltpu.sync_copy(nact, actout.at[pl.ds(base, _RPW)])
    pltpu.sync_copy(cnt_v, counts_out.at[pl.ds(wid * 16, 16)])
    sh.wait()
    for h in gh:
        h.wait()
    pltpu.sync_copy(rows_v, partout.at[pl.ds(base, _RPW)])


def _dot_body(fn_ref, part_ref, cnt_ref, sum_ref, cnt_out_ref):
    x = fn_ref[0:_B, :]
    p = part_ref[...]
    dots = jnp.sum(x * p, axis=1)
    hinge = jnp.maximum(dots - MARGIN, 0.0)
    sum_ref[...] = jnp.broadcast_to(jnp.sum(hinge), (1, 128))
    cnt_out_ref[...] = jnp.broadcast_to(jnp.sum(cnt_ref[...]), (1, 128))


def _tc_dots(fn_ext, partfeat, counts):
    return pl.pallas_call(
        _dot_body,
        in_specs=[
            pl.BlockSpec((_BE, _D), lambda: (0, 0)),
            pl.BlockSpec((_B, _D), lambda: (0, 0)),
            pl.BlockSpec((4, 128), lambda: (0, 0)),
        ],
        out_specs=[
            pl.BlockSpec((1, 128), lambda: (0, 0)),
            pl.BlockSpec((1, 128), lambda: (0, 0)),
        ],
        out_shape=[
            jax.ShapeDtypeStruct((1, 128), jnp.float32),
            jax.ShapeDtypeStruct((1, 128), jnp.int32),
        ],
    )(fn_ext, partfeat, counts)


@jax.jit
def kernel(sem_ids, encoded_features):
    fn_ext, krow = _tc_prep(sem_ids.T, encoded_features)
    keys = krow.reshape(_B)
    rowids = jnp.arange(_B, dtype=jnp.int32)
    tbl0 = _sc_init(keys, rowids)

    def _cond(state):
        return state[4]

    def _body(state):
        act, tbl, total, count, _ = state
        nact, counts, partfeat, ntbl = _sc_round(
            fn_ext, keys, rowids, act, tbl)
        rsum, rcnt = _tc_dots(fn_ext, partfeat, counts.reshape(4, 128))
        pairs = rcnt[0, 0]
        return (nact, ntbl, total + rsum[0, 0], count + pairs, pairs > 0)

    act0 = jnp.ones((_B,), jnp.int32)
    init = (act0, tbl0, jnp.float32(0.0), jnp.int32(0), jnp.bool_(True))
    _, _, total, count, _ = lax.while_loop(_cond, _body, init)
    cf = count.astype(jnp.float32)
    mean = WEIGHT * total / jnp.maximum(cf, 1.0)
    return jnp.where(count > 0, mean, 0.0)
